# baseline (speedup 1.0000x reference)
import functools
import math

import jax
import jax.numpy as jnp
from jax import lax
from jax.experimental import pallas as pl
from jax.experimental.pallas import tpu as pltpu

N_NODES = 10000
N_EDGES = 160000
D_SH = 16
N_BASIS = 10
N_RAD = 64
MAX_R = 3.5
NAVG = 16.0
N_GRAPHS = 16
INV = 1.0 / math.sqrt(NAVG)

BE = 2048


def _silu(v):
    return v * jax.nn.sigmoid(v)


def _edge_chain_body(emb_ref, attr_ref, r0_ref, r1_ref, r2_ref, wsh_ref, s_ref):
    emb = emb_ref[...]
    attr = attr_ref[...]
    u = _silu(emb @ r0_ref[...])
    u = _silu(u @ r1_ref[...])
    g = u @ r2_ref[...]
    a = attr @ wsh_ref[...]
    s_ref[...] = g * (1.0 + a)


def _edge_chain(emb, attr, R0p, R1, R2, Wsh):
    E = emb.shape[0]
    dout = R2.shape[1]
    grid = (E // BE,)
    return pl.pallas_call(
        _edge_chain_body,
        grid=grid,
        in_specs=[
            pl.BlockSpec((BE, 16), lambda i: (i, 0)),
            pl.BlockSpec((BE, 16), lambda i: (i, 0)),
            pl.BlockSpec((16, N_RAD), lambda i: (0, 0)),
            pl.BlockSpec((N_RAD, N_RAD), lambda i: (0, 0)),
            pl.BlockSpec((N_RAD, dout), lambda i: (0, 0)),
            pl.BlockSpec((16, dout), lambda i: (0, 0)),
        ],
        out_specs=pl.BlockSpec((BE, dout), lambda i: (i, 0)),
        out_shape=jax.ShapeDtypeStruct((E, dout), jnp.float32),
    )(emb, attr, R0p, R1, R2, Wsh)


def kernel(pos, x, edge_index, batch, params):
    src = edge_index[0]
    dst = edge_index[1]
    E = src.shape[0]
    Ep = ((E + BE - 1) // BE) * BE

    ev = jnp.take(pos, src, axis=0) - jnp.take(pos, dst, axis=0)
    elen = jnp.sqrt(ev[:, 0] ** 2 + ev[:, 1] ** 2 + ev[:, 2] ** 2)
    unit = ev / (elen[:, None] + 1e-9)
    ux, uy, uz = unit[:, 0], unit[:, 1], unit[:, 2]
    s3 = 3.0 ** 0.5
    s5 = 5.0 ** 0.5
    s15 = 15.0 ** 0.5
    c70 = (70.0 ** 0.5) / 4.0
    c105 = 105.0 ** 0.5
    c42 = (42.0 ** 0.5) / 4.0
    c7 = (7.0 ** 0.5) / 2.0
    c1052 = (105.0 ** 0.5) / 2.0
    sh = jnp.stack([
        jnp.ones_like(ux),
        s3 * ux, s3 * uy, s3 * uz,
        s15 * ux * uy, s15 * uy * uz, (s5 / 2.0) * (3 * uz * uz - 1.0),
        s15 * ux * uz, (s15 / 2.0) * (ux * ux - uy * uy),
        c70 * uy * (3 * ux * ux - uy * uy), c105 * ux * uy * uz,
        c42 * uy * (5 * uz * uz - 1.0), c7 * uz * (5 * uz * uz - 3.0),
        c42 * ux * (5 * uz * uz - 1.0), c1052 * uz * (ux * ux - uy * uy),
        c70 * ux * (ux * ux - 3 * uy * uy),
    ], axis=1)
    centers = jnp.linspace(0.0, MAX_R, N_BASIS)
    step = MAX_R / (N_BASIS - 1)
    emb = jnp.exp(-(((elen[:, None] - centers[None, :]) / step) ** 2)) * (N_BASIS ** 0.5)
    u01 = jnp.clip(elen / MAX_R, 0.0, 1.0)
    cut = jnp.where(elen < MAX_R, 0.5 * (jnp.cos(jnp.pi * u01) + 1.0), 0.0)
    attr = cut[:, None] * sh

    emb16 = jnp.pad(emb, ((0, Ep - E), (0, 16 - N_BASIS)))
    attr16 = jnp.pad(attr, ((0, Ep - E), (0, 0)))

    h = x
    n_layers = len(params)
    for i, p in enumerate(params):
        dout = p['W1'].shape[1]
        y = h @ p['W1']
        z = h @ p['Wsc']
        R0p = jnp.pad(p['R0'], ((0, 16 - N_BASIS), (0, 0)))
        s = _edge_chain(emb16, attr16, R0p, p['R1'], p['R2'], p['Wsh'])[:E]
        m = jnp.take(y, src, axis=0) * s
        agg = jnp.zeros((h.shape[0], dout), jnp.float32).at[dst].add(m) * INV
        h = z + agg
        if i < n_layers - 1:
            h = _silu(h)
    pooled = jax.ops.segment_sum(h, batch, num_segments=N_GRAPHS)
    return (pooled * INV).reshape(-1)


# speedup vs baseline: 2.5925x; 2.5925x over previous
import math

import jax
import jax.numpy as jnp
from jax import lax
from jax.experimental import pallas as pl
from jax.experimental.pallas import tpu as pltpu
from jax.experimental.pallas import tpu_sc as plsc

N_NODES = 10000
N_EDGES = 160000
N_BASIS = 10
N_RAD = 64
MAX_R = 3.5
NAVG = 16.0
N_GRAPHS = 16
INV = 1.0 / math.sqrt(NAVG)

NC = 2
NS = 16
L = 16

BE = 2048
EP = 161792
CHUNK = 128
CHUNK2 = 64
TPS = EP // NS
TPA = EP // (NC * NS)
CH = 128
G2 = 40
NP = 10112
NROW = NP // NS


def _sc_mesh():
    return plsc.VectorSubcoreMesh(core_axis_name="c", subcore_axis_name="s",
                                  num_cores=NC, num_subcores=NS)


def _sc_params():
    return pltpu.CompilerParams(needs_layout_passes=False)


def _silu(v):
    return v * (1.0 / (1.0 + jnp.exp(-v)))


def _edge_vec_body(px_hbm, py_hbm, pz_hbm, src_hbm, dst_hbm,
                   dx_hbm, dy_hbm, dz_hbm,
                   px_v, py_v, pz_v, sv, dv, ox, oy, oz):
    c = lax.axis_index("c")
    sid = lax.axis_index("s")
    wid = sid * NC + c
    base = wid * TPA
    pltpu.sync_copy(px_hbm, px_v)
    pltpu.sync_copy(py_hbm, py_v)
    pltpu.sync_copy(pz_hbm, pz_v)
    pltpu.sync_copy(src_hbm.at[pl.ds(base, TPA)], sv)
    pltpu.sync_copy(dst_hbm.at[pl.ds(base, TPA)], dv)

    def body(i, _):
        sl = pl.ds(i * L, L)
        isrc = sv[sl]
        idst = dv[sl]
        ox[sl] = plsc.load_gather(px_v, [isrc]) - plsc.load_gather(px_v, [idst])
        oy[sl] = plsc.load_gather(py_v, [isrc]) - plsc.load_gather(py_v, [idst])
        oz[sl] = plsc.load_gather(pz_v, [isrc]) - plsc.load_gather(pz_v, [idst])
        return 0

    lax.fori_loop(0, TPA // L, body, 0)
    pltpu.sync_copy(ox, dx_hbm.at[pl.ds(base, TPA)])
    pltpu.sync_copy(oy, dy_hbm.at[pl.ds(base, TPA)])
    pltpu.sync_copy(oz, dz_hbm.at[pl.ds(base, TPA)])


def _edge_vec(px, py, pz, srcp, dstp):
    f32 = jnp.float32
    k = pl.kernel(
        _edge_vec_body,
        out_type=[jax.ShapeDtypeStruct((EP,), f32)] * 3,
        mesh=_sc_mesh(),
        compiler_params=_sc_params(),
        scratch_types=[
            pltpu.VMEM((N_NODES,), f32),
            pltpu.VMEM((N_NODES,), f32),
            pltpu.VMEM((N_NODES,), f32),
            pltpu.VMEM((TPA,), jnp.int32),
            pltpu.VMEM((TPA,), jnp.int32),
            pltpu.VMEM((TPA,), f32),
            pltpu.VMEM((TPA,), f32),
            pltpu.VMEM((TPA,), f32),
        ],
    )
    return k(px, py, pz, srcp, dstp)


def _feat_body(dx_ref, dy_ref, dz_ref, emb_ref, attr_ref):
    dx = dx_ref[...].reshape(1, BE)
    dy = dy_ref[...].reshape(1, BE)
    dz = dz_ref[...].reshape(1, BE)
    r2 = dx * dx + dy * dy + dz * dz
    elen = jnp.sqrt(r2)
    inv = 1.0 / (elen + 1e-9)
    ux = dx * inv
    uy = dy * inv
    uz = dz * inv
    s3 = 3.0 ** 0.5
    s5 = 5.0 ** 0.5
    s15 = 15.0 ** 0.5
    c70 = (70.0 ** 0.5) / 4.0
    c105 = 105.0 ** 0.5
    c42 = (42.0 ** 0.5) / 4.0
    c7 = (7.0 ** 0.5) / 2.0
    c1052 = (105.0 ** 0.5) / 2.0
    sh = jnp.concatenate([
        jnp.ones_like(ux),
        s3 * ux, s3 * uy, s3 * uz,
        s15 * ux * uy, s15 * uy * uz, (s5 / 2.0) * (3 * uz * uz - 1.0),
        s15 * ux * uz, (s15 / 2.0) * (ux * ux - uy * uy),
        c70 * uy * (3 * ux * ux - uy * uy), c105 * ux * uy * uz,
        c42 * uy * (5 * uz * uz - 1.0), c7 * uz * (5 * uz * uz - 3.0),
        c42 * ux * (5 * uz * uz - 1.0), c1052 * uz * (ux * ux - uy * uy),
        c70 * ux * (ux * ux - 3 * uy * uy),
    ], axis=0)
    step = MAX_R / (N_BASIS - 1)
    centers = lax.broadcasted_iota(jnp.int32, (16, 1), 0).astype(jnp.float32) * step
    emb = jnp.exp(-(((elen - centers) / step) ** 2)) * (N_BASIS ** 0.5)
    u01 = jnp.clip(elen / MAX_R, 0.0, 1.0)
    cut = jnp.where(elen < MAX_R, 0.5 * (jnp.cos(jnp.pi * u01) + 1.0), 0.0)
    attr = cut * sh
    emb_ref[...] = emb.T
    attr_ref[...] = attr.T


def _features(dx, dy, dz):
    return pl.pallas_call(
        _feat_body,
        grid=(EP // BE,),
        in_specs=[pl.BlockSpec((BE,), lambda i: (i,))] * 3,
        out_specs=[pl.BlockSpec((BE, 16), lambda i: (i, 0))] * 2,
        out_shape=[jax.ShapeDtypeStruct((EP, 16), jnp.float32)] * 2,
    )(dx, dy, dz)


BN = 400


def _combine(z_ref, a0_ref, a1_ref, p0_ref, p1_ref):
    g2 = (p0_ref[...] + p1_ref[...])[:, :G2]
    agg = jnp.concatenate([a0_ref[...], a1_ref[...], g2], axis=1)
    return _silu(z_ref[...] + agg * INV)


def _node_mm_first_body(x_ref, w0_ref, w1_ref, w2_ref, wsc_ref,
                        y0_ref, y1_ref, y2_ref, z_ref):
    xf = x_ref[...]
    y0_ref[...] = jnp.dot(xf, w0_ref[...], preferred_element_type=jnp.float32, precision=lax.Precision.HIGHEST)
    y1_ref[...] = jnp.dot(xf, w1_ref[...], preferred_element_type=jnp.float32, precision=lax.Precision.HIGHEST)
    y2_ref[...] = jnp.dot(xf, w2_ref[...], preferred_element_type=jnp.float32, precision=lax.Precision.HIGHEST)
    z_ref[...] = jnp.dot(xf, wsc_ref[...], preferred_element_type=jnp.float32, precision=lax.Precision.HIGHEST)


def _node_mm_mid_body(z_ref, a0_ref, a1_ref, p0_ref, p1_ref,
                      w0_ref, w1_ref, w2_ref, wsc_ref,
                      y0_ref, y1_ref, y2_ref, zo_ref):
    xf = _combine(z_ref, a0_ref, a1_ref, p0_ref, p1_ref)
    y0_ref[...] = jnp.dot(xf, w0_ref[...], preferred_element_type=jnp.float32, precision=lax.Precision.HIGHEST)
    y1_ref[...] = jnp.dot(xf, w1_ref[...], preferred_element_type=jnp.float32, precision=lax.Precision.HIGHEST)
    y2_ref[...] = jnp.dot(xf, w2_ref[...], preferred_element_type=jnp.float32, precision=lax.Precision.HIGHEST)
    zo_ref[...] = jnp.dot(xf, wsc_ref[...], preferred_element_type=jnp.float32, precision=lax.Precision.HIGHEST)


def _node_mm(xf_or_z, w0, w1, w2, wsc, aggs=None):
    din = w0.shape[0]
    nin = [pl.BlockSpec((BN, din), lambda i: (i, 0))]
    args = [xf_or_z]
    body = _node_mm_first_body
    if aggs is not None:
        nin += [pl.BlockSpec((BN, CH), lambda i: (i, 0))] * 4
        args += list(aggs)
        body = _node_mm_mid_body
    nin += [
        pl.BlockSpec((din, CH), lambda i: (0, 0)),
        pl.BlockSpec((din, CH), lambda i: (0, 0)),
        pl.BlockSpec((din, CH), lambda i: (0, 0)),
        pl.BlockSpec((din, 296), lambda i: (0, 0)),
    ]
    args += [w0, w1, w2, wsc]
    return pl.pallas_call(
        body,
        grid=(N_NODES // BN,),
        in_specs=nin,
        out_specs=[
            pl.BlockSpec((BN, CH), lambda i: (i, 0)),
            pl.BlockSpec((BN, CH), lambda i: (i, 0)),
            pl.BlockSpec((BN, CH), lambda i: (i, 0)),
            pl.BlockSpec((BN, 296), lambda i: (i, 0)),
        ],
        out_shape=[
            jax.ShapeDtypeStruct((N_NODES, CH), jnp.float32),
            jax.ShapeDtypeStruct((N_NODES, CH), jnp.float32),
            jax.ShapeDtypeStruct((N_NODES, CH), jnp.float32),
            jax.ShapeDtypeStruct((N_NODES, 296), jnp.float32),
        ],
    )(*args)


def _node_mm_last_body(z_ref, a0_ref, a1_ref, p0_ref, p1_ref, w4_ref,
                       y4_ref, xf_ref):
    xf = _combine(z_ref, a0_ref, a1_ref, p0_ref, p1_ref)
    y4_ref[...] = jnp.dot(xf, w4_ref[...], preferred_element_type=jnp.float32, precision=lax.Precision.HIGHEST)
    xf_ref[...] = xf


def _node_mm_last(z, aggs, w4pad):
    return pl.pallas_call(
        _node_mm_last_body,
        grid=(N_NODES // BN,),
        in_specs=[
            pl.BlockSpec((BN, 296), lambda i: (i, 0)),
            pl.BlockSpec((BN, CH), lambda i: (i, 0)),
            pl.BlockSpec((BN, CH), lambda i: (i, 0)),
            pl.BlockSpec((BN, CH), lambda i: (i, 0)),
            pl.BlockSpec((BN, CH), lambda i: (i, 0)),
            pl.BlockSpec((296, 128), lambda i: (0, 0)),
        ],
        out_specs=[
            pl.BlockSpec((BN, 128), lambda i: (i, 0)),
            pl.BlockSpec((BN, 296), lambda i: (i, 0)),
        ],
        out_shape=[
            jax.ShapeDtypeStruct((N_NODES, 128), jnp.float32),
            jax.ShapeDtypeStruct((N_NODES, 296), jnp.float32),
        ],
    )(z, *aggs, w4pad)


def _edge_chain_body(nout, emb_ref, attr_ref, r0_ref, r1_ref, *rest):
    r2_refs = rest[:nout]
    wsh_refs = rest[nout:2 * nout]
    s_refs = rest[2 * nout:]
    i = pl.program_id(0)
    emb = emb_ref[...]
    attr = attr_ref[...]
    u = _silu(jnp.dot(emb, r0_ref[...], preferred_element_type=jnp.float32, precision=lax.Precision.HIGHEST))
    u = _silu(jnp.dot(u, r1_ref[...], preferred_element_type=jnp.float32, precision=lax.Precision.HIGHEST))
    rows = i * BE + lax.broadcasted_iota(jnp.int32, (BE, 1), 0)
    mask = (rows < N_EDGES).astype(jnp.float32)
    for r2_ref, wsh_ref, s_ref in zip(r2_refs, wsh_refs, s_refs):
        g = jnp.dot(u, r2_ref[...], preferred_element_type=jnp.float32, precision=lax.Precision.HIGHEST)
        a = jnp.dot(attr, wsh_ref[...], preferred_element_type=jnp.float32, precision=lax.Precision.HIGHEST)
        s_ref[...] = g * (1.0 + a) * mask


def _edge_chain(emb, attr, r0p, r1, r2s, wshs):
    nout = len(r2s)
    body = lambda *refs: _edge_chain_body(nout, *refs)
    return pl.pallas_call(
        body,
        grid=(EP // BE,),
        in_specs=[
            pl.BlockSpec((BE, 16), lambda i: (i, 0)),
            pl.BlockSpec((BE, 16), lambda i: (i, 0)),
            pl.BlockSpec((16, N_RAD), lambda i: (0, 0)),
            pl.BlockSpec((N_RAD, N_RAD), lambda i: (0, 0)),
        ] + [pl.BlockSpec((N_RAD, CH), lambda i: (0, 0))] * nout
          + [pl.BlockSpec((16, CH), lambda i: (0, 0))] * nout,
        out_specs=[pl.BlockSpec((BE, CH), lambda i: (i, 0))] * nout,
        out_shape=[jax.ShapeDtypeStruct((EP, CH), jnp.float32)] * nout,
    )(emb, attr, r0p, r1, *r2s, *wshs)


def _edge_mp_body(y0_hbm, y1_hbm, s0_hbm, s1_hbm, src_hbm, dst_hbm, zc_hbm,
                  a0_hbm, a1_hbm,
                  isv, idv, rows, srow, acc, sem):
    c = lax.axis_index("c")
    sid = lax.axis_index("s")
    base = sid * TPS
    pltpu.sync_copy(zc_hbm.at[pl.ds(sid * NROW, NROW)],
                    acc.at[pl.ds(sid * NROW, NROW)])
    plsc.subcore_barrier()

    def run(y_hbm, s_hbm, a_hbm):
        def chunk(k, _):
            e0 = base + k * CHUNK
            pltpu.sync_copy(src_hbm.at[pl.ds(e0, CHUNK)], isv)
            pltpu.sync_copy(dst_hbm.at[pl.ds(e0, CHUNK)], idv)
            cp = pltpu.async_copy(y_hbm.at[isv], rows, sem)
            pltpu.sync_copy(s_hbm.at[pl.ds(e0, CHUNK)], srow)
            cp.wait()

            def mul(i, _):
                for j in range(CH // L):
                    sl = pl.ds(j * L, L)
                    rows[i, sl] = rows[i, sl] * srow[i, sl]
                return 0

            lax.fori_loop(0, CHUNK, mul, 0)
            pltpu.sync_copy(rows, acc.at[idv], add=True)
            return 0

        lax.fori_loop(0, TPS // CHUNK, chunk, 0)
        plsc.subcore_barrier()
        pltpu.sync_copy(acc.at[pl.ds(sid * NROW, NROW)],
                        a_hbm.at[pl.ds(sid * NROW, NROW)])

    @pl.when(c == 0)
    def _():
        run(y0_hbm, s0_hbm, a0_hbm)

    @pl.when(c == 1)
    def _():
        run(y1_hbm, s1_hbm, a1_hbm)


def _edge_mp(y0, y1, s0, s1, srcp, dstp, zc):
    f32 = jnp.float32
    k = pl.kernel(
        _edge_mp_body,
        out_type=[jax.ShapeDtypeStruct((NP, CH), f32)] * 2,
        mesh=_sc_mesh(),
        compiler_params=_sc_params(),
        scratch_types=[
            pltpu.VMEM((CHUNK,), jnp.int32),
            pltpu.VMEM((CHUNK,), jnp.int32),
            pltpu.VMEM((CHUNK, CH), f32),
            pltpu.VMEM((CHUNK, CH), f32),
            pltpu.VMEM_SHARED((NP, CH), f32),
            pltpu.SemaphoreType.DMA,
        ],
    )
    return k(y0, y1, s0, s1, srcp, dstp, zc)


def _edge_mp2_body(y_hbm, s_hbm, src_hbm, dst_hbm, zc_hbm, p0_hbm, p1_hbm,
                   isv, idv, rows, srow, acc, sem):
    c = lax.axis_index("c")
    sid = lax.axis_index("s")
    base = (c * NS + sid) * TPA
    pltpu.sync_copy(zc_hbm.at[pl.ds(sid * NROW, NROW)],
                    acc.at[pl.ds(sid * NROW, NROW)])
    plsc.subcore_barrier()

    def chunk(k, _):
        e0 = base + k * CHUNK2
        pltpu.sync_copy(src_hbm.at[pl.ds(e0, CHUNK2)], isv)
        pltpu.sync_copy(dst_hbm.at[pl.ds(e0, CHUNK2)], idv)
        cp = pltpu.async_copy(y_hbm.at[isv], rows, sem)
        pltpu.sync_copy(s_hbm.at[pl.ds(e0, CHUNK2)], srow)
        cp.wait()

        def mul(i, _):
            for j in range(CH // L):
                sl = pl.ds(j * L, L)
                rows[i, sl] = rows[i, sl] * srow[i, sl]
            return 0

        lax.fori_loop(0, CHUNK2, mul, 0)
        pltpu.sync_copy(rows, acc.at[idv], add=True)
        return 0

    lax.fori_loop(0, TPA // CHUNK2, chunk, 0)
    plsc.subcore_barrier()

    @pl.when(c == 0)
    def _():
        pltpu.sync_copy(acc.at[pl.ds(sid * NROW, NROW)],
                        p0_hbm.at[pl.ds(sid * NROW, NROW)])

    @pl.when(c == 1)
    def _():
        pltpu.sync_copy(acc.at[pl.ds(sid * NROW, NROW)],
                        p1_hbm.at[pl.ds(sid * NROW, NROW)])


def _edge_mp2(y2, s2, srcp, dstp, zc):
    f32 = jnp.float32
    k = pl.kernel(
        _edge_mp2_body,
        out_type=[jax.ShapeDtypeStruct((NP, CH), f32)] * 2,
        mesh=_sc_mesh(),
        compiler_params=_sc_params(),
        scratch_types=[
            pltpu.VMEM((CHUNK2,), jnp.int32),
            pltpu.VMEM((CHUNK2,), jnp.int32),
            pltpu.VMEM((CHUNK2, CH), f32),
            pltpu.VMEM((CHUNK2, CH), f32),
            pltpu.VMEM_SHARED((NP, CH), f32),
            pltpu.SemaphoreType.DMA,
        ],
    )
    return k(y2, s2, srcp, dstp, zc)


def _edge_last_body(y4_hbm, bt_hbm, s4_hbm, src_hbm, dst_hbm, bins_hbm,
                    ytab, btab, sv, dv, s4v, bins, binsum):
    c = lax.axis_index("c")
    sid = lax.axis_index("s")
    wid = sid * NC + c
    base = wid * TPA

    def zb(i, _):
        bins[pl.ds(i * L, L)] = jnp.zeros((L,), jnp.float32)
        return 0

    lax.fori_loop(0, (L * N_GRAPHS) // L, zb, 0)
    pltpu.sync_copy(y4_hbm, ytab)
    pltpu.sync_copy(bt_hbm, btab)
    pltpu.sync_copy(src_hbm.at[pl.ds(base, TPA)], sv)
    pltpu.sync_copy(dst_hbm.at[pl.ds(base, TPA)], dv)
    pltpu.sync_copy(s4_hbm.at[pl.ds(base, TPA)], s4v)
    lanes = lax.iota(jnp.int32, L)

    def body(i, _):
        sl = pl.ds(i * L, L)
        isrc = sv[sl]
        idst = dv[sl]
        yv = plsc.load_gather(ytab, [isrc])
        bv = plsc.load_gather(btab, [idst])
        m = yv * s4v[sl]
        plsc.addupdate_scatter(bins, [lanes * N_GRAPHS + bv], m)
        return 0

    lax.fori_loop(0, TPA // L, body, 0)
    acc = bins[pl.ds(0, L)]
    for l in range(1, L):
        acc = acc + bins[pl.ds(l * L, L)]
    binsum[pl.ds(0, L)] = acc
    pltpu.sync_copy(binsum, bins_hbm.at[pl.ds(wid * N_GRAPHS, N_GRAPHS)])


def _edge_last(y4f, batch_i32, s4f, srcp, dstp):
    f32 = jnp.float32
    k = pl.kernel(
        _edge_last_body,
        out_type=[jax.ShapeDtypeStruct((NC * NS * N_GRAPHS,), f32)],
        mesh=_sc_mesh(),
        compiler_params=_sc_params(),
        scratch_types=[
            pltpu.VMEM((N_NODES,), f32),
            pltpu.VMEM((N_NODES,), jnp.int32),
            pltpu.VMEM((TPA,), jnp.int32),
            pltpu.VMEM((TPA,), jnp.int32),
            pltpu.VMEM((TPA,), f32),
            pltpu.VMEM((L * N_GRAPHS,), f32),
            pltpu.VMEM((N_GRAPHS,), f32),
        ],
    )
    return k(y4f, batch_i32, s4f, srcp, dstp)


def _pool_body(xf_ref, b_ref, bins_ref, wsc_ref, out_ref):
    i = pl.program_id(0)
    xf = xf_ref[...]
    b = b_ref[...]
    onehot = (b[:, 0:1] == lax.broadcasted_iota(jnp.int32, (1, N_GRAPHS), 1))
    ps = jnp.dot(onehot.astype(jnp.float32).T, xf,
                 preferred_element_type=jnp.float32, precision=lax.Precision.HIGHEST)
    zsum = jnp.dot(ps, wsc_ref[...],
                   preferred_element_type=jnp.float32, precision=lax.Precision.HIGHEST)

    @pl.when(i == 0)
    def _():
        bsum = jnp.sum(bins_ref[...], axis=0, keepdims=True)
        out_ref[...] = bsum * (INV * INV)

    out_ref[...] += zsum.T * INV


def _pool(xf, batch2d, bins, wsc4):
    return pl.pallas_call(
        _pool_body,
        grid=(N_NODES // BN,),
        in_specs=[
            pl.BlockSpec((BN, 296), lambda i: (i, 0)),
            pl.BlockSpec((BN, 128), lambda i: (i, 0)),
            pl.BlockSpec((NC * NS, N_GRAPHS), lambda i: (0, 0)),
            pl.BlockSpec((296, 1), lambda i: (0, 0)),
        ],
        out_specs=pl.BlockSpec((1, N_GRAPHS), lambda i: (0, 0)),
        out_shape=jax.ShapeDtypeStruct((1, N_GRAPHS), jnp.float32),
    )(xf, batch2d, bins, wsc4)


def _split3(w):
    return (w[:, :CH], w[:, CH:2 * CH],
            jnp.pad(w[:, 2 * CH:], ((0, 0), (0, CH - G2))))


def kernel(pos, x, edge_index, batch, params):
    f32 = jnp.float32
    src = edge_index[0].astype(jnp.int32)
    dst = edge_index[1].astype(jnp.int32)
    srcp = jnp.pad(src, (0, EP - N_EDGES))
    dstp = jnp.pad(dst, (0, EP - N_EDGES))
    px = pos[:, 0].astype(f32)
    py = pos[:, 1].astype(f32)
    pz = pos[:, 2].astype(f32)

    dx, dy, dz = _edge_vec(px, py, pz, srcp, dstp)
    emb, attr = _features(dx, dy, dz)
    zc = jnp.zeros((NP, CH), f32)

    aggs = None
    zprev = None
    for i in range(3):
        p = params[i]
        r0p = jnp.pad(p['R0'], ((0, 16 - N_BASIS), (0, 0)))
        w0, w1, w2 = _split3(p['W1'])
        r2s = _split3(p['R2'])
        wshs = _split3(p['Wsh'])
        if i == 0:
            y0, y1, y2, z = _node_mm(x, w0, w1, w2, p['Wsc'])
        else:
            y0, y1, y2, z = _node_mm(zprev, w0, w1, w2, p['Wsc'], aggs)
        s0, s1, s2 = _edge_chain(emb, attr, r0p, p['R1'], r2s, wshs)
        a0, a1 = _edge_mp(y0, y1, s0, s1, srcp, dstp, zc)
        p0, p1 = _edge_mp2(y2, s2, srcp, dstp, zc)
        aggs = (a0, a1, p0, p1)
        zprev = z

    p = params[3]
    r0p = jnp.pad(p['R0'], ((0, 16 - N_BASIS), (0, 0)))
    r2c = jnp.pad(p['R2'], ((0, 0), (0, CH - 1)))
    wshc = jnp.pad(p['Wsh'], ((0, 0), (0, CH - 1)))
    w4pad = jnp.pad(p['W1'], ((0, 0), (0, 127)))
    (s4a,) = _edge_chain(emb, attr, r0p, p['R1'], [r2c], [wshc])
    s4f = s4a[:, 0]
    y4mat, xf3 = _node_mm_last(zprev, aggs, w4pad)
    y4f = y4mat[:, 0]
    (bins,) = _edge_last(y4f, batch.astype(jnp.int32), s4f, srcp, dstp)
    bins2d = bins.reshape(NC * NS, N_GRAPHS)
    batch2d = jnp.pad(batch.astype(jnp.int32).reshape(N_NODES, 1),
                      ((0, 0), (0, 127)))
    out = _pool(xf3, batch2d, bins2d, p['Wsc'])
    return out.reshape(-1)


# speedup vs baseline: 3.0509x; 1.1768x over previous
import math

import jax
import jax.numpy as jnp
from jax import lax
from jax.experimental import pallas as pl
from jax.experimental.pallas import tpu as pltpu
from jax.experimental.pallas import tpu_sc as plsc

N_NODES = 10000
N_EDGES = 160000
N_BASIS = 10
N_RAD = 64
MAX_R = 3.5
NAVG = 16.0
N_GRAPHS = 16
INV = 1.0 / math.sqrt(NAVG)

NC = 2
NS = 16
L = 16

BE = 2048
EP = 161792
CHUNK = 64
TPS = EP // NS
TPA = EP // (NC * NS)
CH = 128
G2 = 40
NP = 10112
NROW = NP // NS


def _sc_mesh():
    return plsc.VectorSubcoreMesh(core_axis_name="c", subcore_axis_name="s",
                                  num_cores=NC, num_subcores=NS)


def _sc_params():
    return pltpu.CompilerParams(needs_layout_passes=False)


def _silu(v):
    return v * (1.0 / (1.0 + jnp.exp(-v)))


def _edge_vec_body(px_hbm, py_hbm, pz_hbm, src_hbm, dst_hbm,
                   dx_hbm, dy_hbm, dz_hbm,
                   px_v, py_v, pz_v, sv, dv, ox, oy, oz):
    c = lax.axis_index("c")
    sid = lax.axis_index("s")
    wid = sid * NC + c
    base = wid * TPA
    pltpu.sync_copy(px_hbm, px_v)
    pltpu.sync_copy(py_hbm, py_v)
    pltpu.sync_copy(pz_hbm, pz_v)
    pltpu.sync_copy(src_hbm.at[pl.ds(base, TPA)], sv)
    pltpu.sync_copy(dst_hbm.at[pl.ds(base, TPA)], dv)

    def body(i, _):
        sl = pl.ds(i * L, L)
        isrc = sv[sl]
        idst = dv[sl]
        ox[sl] = plsc.load_gather(px_v, [isrc]) - plsc.load_gather(px_v, [idst])
        oy[sl] = plsc.load_gather(py_v, [isrc]) - plsc.load_gather(py_v, [idst])
        oz[sl] = plsc.load_gather(pz_v, [isrc]) - plsc.load_gather(pz_v, [idst])
        return 0

    lax.fori_loop(0, TPA // L, body, 0)
    pltpu.sync_copy(ox, dx_hbm.at[pl.ds(base, TPA)])
    pltpu.sync_copy(oy, dy_hbm.at[pl.ds(base, TPA)])
    pltpu.sync_copy(oz, dz_hbm.at[pl.ds(base, TPA)])


def _edge_vec(px, py, pz, srcp, dstp):
    f32 = jnp.float32
    k = pl.kernel(
        _edge_vec_body,
        out_type=[jax.ShapeDtypeStruct((EP,), f32)] * 3,
        mesh=_sc_mesh(),
        compiler_params=_sc_params(),
        scratch_types=[
            pltpu.VMEM((N_NODES,), f32),
            pltpu.VMEM((N_NODES,), f32),
            pltpu.VMEM((N_NODES,), f32),
            pltpu.VMEM((TPA,), jnp.int32),
            pltpu.VMEM((TPA,), jnp.int32),
            pltpu.VMEM((TPA,), f32),
            pltpu.VMEM((TPA,), f32),
            pltpu.VMEM((TPA,), f32),
        ],
    )
    return k(px, py, pz, srcp, dstp)


def _feat_body(dx_ref, dy_ref, dz_ref, emb_ref, attr_ref):
    dx = dx_ref[...].reshape(1, BE)
    dy = dy_ref[...].reshape(1, BE)
    dz = dz_ref[...].reshape(1, BE)
    r2 = dx * dx + dy * dy + dz * dz
    elen = jnp.sqrt(r2)
    inv = 1.0 / (elen + 1e-9)
    ux = dx * inv
    uy = dy * inv
    uz = dz * inv
    s3 = 3.0 ** 0.5
    s5 = 5.0 ** 0.5
    s15 = 15.0 ** 0.5
    c70 = (70.0 ** 0.5) / 4.0
    c105 = 105.0 ** 0.5
    c42 = (42.0 ** 0.5) / 4.0
    c7 = (7.0 ** 0.5) / 2.0
    c1052 = (105.0 ** 0.5) / 2.0
    sh = jnp.concatenate([
        jnp.ones_like(ux),
        s3 * ux, s3 * uy, s3 * uz,
        s15 * ux * uy, s15 * uy * uz, (s5 / 2.0) * (3 * uz * uz - 1.0),
        s15 * ux * uz, (s15 / 2.0) * (ux * ux - uy * uy),
        c70 * uy * (3 * ux * ux - uy * uy), c105 * ux * uy * uz,
        c42 * uy * (5 * uz * uz - 1.0), c7 * uz * (5 * uz * uz - 3.0),
        c42 * ux * (5 * uz * uz - 1.0), c1052 * uz * (ux * ux - uy * uy),
        c70 * ux * (ux * ux - 3 * uy * uy),
    ], axis=0)
    step = MAX_R / (N_BASIS - 1)
    centers = lax.broadcasted_iota(jnp.int32, (16, 1), 0).astype(jnp.float32) * step
    emb = jnp.exp(-(((elen - centers) / step) ** 2)) * (N_BASIS ** 0.5)
    u01 = jnp.clip(elen / MAX_R, 0.0, 1.0)
    cut = jnp.where(elen < MAX_R, 0.5 * (jnp.cos(jnp.pi * u01) + 1.0), 0.0)
    attr = cut * sh
    emb_ref[...] = emb.T
    attr_ref[...] = attr.T


def _features(dx, dy, dz):
    return pl.pallas_call(
        _feat_body,
        grid=(EP // BE,),
        in_specs=[pl.BlockSpec((BE,), lambda i: (i,))] * 3,
        out_specs=[pl.BlockSpec((BE, 16), lambda i: (i, 0))] * 2,
        out_shape=[jax.ShapeDtypeStruct((EP, 16), jnp.float32)] * 2,
    )(dx, dy, dz)


BN = 400


def _combine(z_ref, a0_ref, a1_ref, p0_ref, p1_ref):
    g2 = (p0_ref[...] + p1_ref[...])[:, :G2]
    agg = jnp.concatenate([a0_ref[...], a1_ref[...], g2], axis=1)
    return _silu(z_ref[...] + agg * INV)


def _node_mm_first_body(x_ref, w0_ref, w1_ref, w2_ref, wsc_ref,
                        y0_ref, y1_ref, y2_ref, z_ref):
    xf = x_ref[...]
    y0_ref[...] = jnp.dot(xf, w0_ref[...], preferred_element_type=jnp.float32, precision=lax.Precision.HIGHEST)
    y1_ref[...] = jnp.dot(xf, w1_ref[...], preferred_element_type=jnp.float32, precision=lax.Precision.HIGHEST)
    y2_ref[...] = jnp.dot(xf, w2_ref[...], preferred_element_type=jnp.float32, precision=lax.Precision.HIGHEST)
    z_ref[...] = jnp.dot(xf, wsc_ref[...], preferred_element_type=jnp.float32, precision=lax.Precision.HIGHEST)


def _node_mm_mid_body(z_ref, a0_ref, a1_ref, p0_ref, p1_ref,
                      w0_ref, w1_ref, w2_ref, wsc_ref,
                      y0_ref, y1_ref, y2_ref, zo_ref):
    xf = _combine(z_ref, a0_ref, a1_ref, p0_ref, p1_ref)
    y0_ref[...] = jnp.dot(xf, w0_ref[...], preferred_element_type=jnp.float32, precision=lax.Precision.HIGHEST)
    y1_ref[...] = jnp.dot(xf, w1_ref[...], preferred_element_type=jnp.float32, precision=lax.Precision.HIGHEST)
    y2_ref[...] = jnp.dot(xf, w2_ref[...], preferred_element_type=jnp.float32, precision=lax.Precision.HIGHEST)
    zo_ref[...] = jnp.dot(xf, wsc_ref[...], preferred_element_type=jnp.float32, precision=lax.Precision.HIGHEST)


def _node_mm(xf_or_z, w0, w1, w2, wsc, aggs=None):
    din = w0.shape[0]
    nin = [pl.BlockSpec((BN, din), lambda i: (i, 0))]
    args = [xf_or_z]
    body = _node_mm_first_body
    if aggs is not None:
        nin += [pl.BlockSpec((BN, CH), lambda i: (i, 0))] * 4
        args += list(aggs)
        body = _node_mm_mid_body
    nin += [
        pl.BlockSpec((din, CH), lambda i: (0, 0)),
        pl.BlockSpec((din, CH), lambda i: (0, 0)),
        pl.BlockSpec((din, CH), lambda i: (0, 0)),
        pl.BlockSpec((din, 296), lambda i: (0, 0)),
    ]
    args += [w0, w1, w2, wsc]
    return pl.pallas_call(
        body,
        grid=(N_NODES // BN,),
        in_specs=nin,
        out_specs=[
            pl.BlockSpec((BN, CH), lambda i: (i, 0)),
            pl.BlockSpec((BN, CH), lambda i: (i, 0)),
            pl.BlockSpec((BN, CH), lambda i: (i, 0)),
            pl.BlockSpec((BN, 296), lambda i: (i, 0)),
        ],
        out_shape=[
            jax.ShapeDtypeStruct((N_NODES, CH), jnp.float32),
            jax.ShapeDtypeStruct((N_NODES, CH), jnp.float32),
            jax.ShapeDtypeStruct((N_NODES, CH), jnp.float32),
            jax.ShapeDtypeStruct((N_NODES, 296), jnp.float32),
        ],
    )(*args)


def _node_mm_last_body(z_ref, a0_ref, a1_ref, p0_ref, p1_ref, w4_ref,
                       y4_ref, xf_ref):
    xf = _combine(z_ref, a0_ref, a1_ref, p0_ref, p1_ref)
    y4_ref[...] = jnp.dot(xf, w4_ref[...], preferred_element_type=jnp.float32, precision=lax.Precision.HIGHEST)
    xf_ref[...] = xf


def _node_mm_last(z, aggs, w4pad):
    return pl.pallas_call(
        _node_mm_last_body,
        grid=(N_NODES // BN,),
        in_specs=[
            pl.BlockSpec((BN, 296), lambda i: (i, 0)),
            pl.BlockSpec((BN, CH), lambda i: (i, 0)),
            pl.BlockSpec((BN, CH), lambda i: (i, 0)),
            pl.BlockSpec((BN, CH), lambda i: (i, 0)),
            pl.BlockSpec((BN, CH), lambda i: (i, 0)),
            pl.BlockSpec((296, 128), lambda i: (0, 0)),
        ],
        out_specs=[
            pl.BlockSpec((BN, 128), lambda i: (i, 0)),
            pl.BlockSpec((BN, 296), lambda i: (i, 0)),
        ],
        out_shape=[
            jax.ShapeDtypeStruct((N_NODES, 128), jnp.float32),
            jax.ShapeDtypeStruct((N_NODES, 296), jnp.float32),
        ],
    )(z, *aggs, w4pad)


def _edge_chain_body(nout, emb_ref, attr_ref, r0_ref, r1_ref, *rest):
    r2_refs = rest[:nout]
    wsh_refs = rest[nout:2 * nout]
    s_refs = rest[2 * nout:]
    i = pl.program_id(0)
    emb = emb_ref[...]
    attr = attr_ref[...]
    u = _silu(jnp.dot(emb, r0_ref[...], preferred_element_type=jnp.float32, precision=lax.Precision.HIGHEST))
    u = _silu(jnp.dot(u, r1_ref[...], preferred_element_type=jnp.float32, precision=lax.Precision.HIGHEST))
    rows = i * BE + lax.broadcasted_iota(jnp.int32, (BE, 1), 0)
    mask = (rows < N_EDGES).astype(jnp.float32)
    for r2_ref, wsh_ref, s_ref in zip(r2_refs, wsh_refs, s_refs):
        g = jnp.dot(u, r2_ref[...], preferred_element_type=jnp.float32, precision=lax.Precision.HIGHEST)
        a = jnp.dot(attr, wsh_ref[...], preferred_element_type=jnp.float32, precision=lax.Precision.HIGHEST)
        s_ref[...] = g * (1.0 + a) * mask


def _edge_chain(emb, attr, r0p, r1, r2s, wshs):
    nout = len(r2s)
    body = lambda *refs: _edge_chain_body(nout, *refs)
    return pl.pallas_call(
        body,
        grid=(EP // BE,),
        in_specs=[
            pl.BlockSpec((BE, 16), lambda i: (i, 0)),
            pl.BlockSpec((BE, 16), lambda i: (i, 0)),
            pl.BlockSpec((16, N_RAD), lambda i: (0, 0)),
            pl.BlockSpec((N_RAD, N_RAD), lambda i: (0, 0)),
        ] + [pl.BlockSpec((N_RAD, CH), lambda i: (0, 0))] * nout
          + [pl.BlockSpec((16, CH), lambda i: (0, 0))] * nout,
        out_specs=[pl.BlockSpec((BE, CH), lambda i: (i, 0))] * nout,
        out_shape=[jax.ShapeDtypeStruct((EP, CH), jnp.float32)] * nout,
    )(emb, attr, r0p, r1, *r2s, *wshs)


def _mp_pipeline(y_hbm, s_hbm, src_hbm, dst_hbm, acc, isv, idv, rows, srow,
                 semi, semj, semg, sems, semw, nchunk, csz, ebase):
    def issue(k, b):
        e0 = ebase + k * csz
        pltpu.async_copy(src_hbm.at[pl.ds(e0, csz)], isv[b], semi[b])
        pltpu.async_copy(dst_hbm.at[pl.ds(e0, csz)], idv[b], semj[b])
        pltpu.async_copy(s_hbm.at[pl.ds(e0, csz)], srow[b], sems[b])
        pltpu.make_async_copy(src_hbm.at[pl.ds(e0, csz)], isv[b], semi[b]).wait()
        pltpu.async_copy(y_hbm.at[isv[b]], rows[b], semg[b])

    def drain(k, b):
        e0 = ebase + k * csz
        pltpu.make_async_copy(y_hbm.at[isv[b]], rows[b], semg[b]).wait()
        pltpu.make_async_copy(s_hbm.at[pl.ds(e0, csz)], srow[b], sems[b]).wait()
        pltpu.make_async_copy(dst_hbm.at[pl.ds(e0, csz)], idv[b], semj[b]).wait()

        def mul(i, _):
            for j in range(CH // L):
                sl = pl.ds(j * L, L)
                rows[b][i, sl] = rows[b][i, sl] * srow[b][i, sl]
            return 0

        lax.fori_loop(0, csz, mul, 0)
        pltpu.async_copy(rows[b], acc.at[idv[b]], semw[b], add=True)

    issue(0, 0)
    issue(1, 1)

    def step(kk, _):
        for b in range(2):
            k = 2 * kk + b

            @pl.when(k < nchunk)
            def _():
                drain(k, b)

            @pl.when(k + 2 < nchunk)
            def _():
                pltpu.make_async_copy(rows[b], acc.at[idv[b]], semw[b]).wait()
                issue(k + 2, b)
        return 0

    lax.fori_loop(0, (nchunk + 1) // 2, step, 0)
    for b in range(2):
        pltpu.make_async_copy(rows[b], acc.at[idv[b]], semw[b]).wait()


def _sc_mp_scratch():
    f32 = jnp.float32
    return ([pltpu.VMEM((CHUNK,), jnp.int32)] * 4
            + [pltpu.VMEM((CHUNK, CH), f32)] * 4
            + [pltpu.VMEM_SHARED((NP, CH), f32)]
            + [pltpu.SemaphoreType.DMA] * 10)


def _edge_mp_body(y0_hbm, y1_hbm, s0_hbm, s1_hbm, src_hbm, dst_hbm, zc_hbm,
                  a0_hbm, a1_hbm,
                  isv0, isv1, idv0, idv1, rows0, rows1, srow0, srow1, acc,
                  semi0, semi1, semj0, semj1, semg0, semg1, sems0, sems1,
                  semw0, semw1):
    c = lax.axis_index("c")
    sid = lax.axis_index("s")
    pltpu.sync_copy(zc_hbm.at[pl.ds(sid * NROW, NROW)],
                    acc.at[pl.ds(sid * NROW, NROW)])
    plsc.subcore_barrier()
    nchunk = TPS // CHUNK

    def run(y_hbm, s_hbm, a_hbm):
        _mp_pipeline(y_hbm, s_hbm, src_hbm, dst_hbm, acc,
                     (isv0, isv1), (idv0, idv1), (rows0, rows1), (srow0, srow1),
                     (semi0, semi1), (semj0, semj1), (semg0, semg1),
                     (sems0, sems1), (semw0, semw1),
                     nchunk, CHUNK, sid * TPS)
        plsc.subcore_barrier()
        pltpu.sync_copy(acc.at[pl.ds(sid * NROW, NROW)],
                        a_hbm.at[pl.ds(sid * NROW, NROW)])

    @pl.when(c == 0)
    def _():
        run(y0_hbm, s0_hbm, a0_hbm)

    @pl.when(c == 1)
    def _():
        run(y1_hbm, s1_hbm, a1_hbm)


def _edge_mp(y0, y1, s0, s1, srcp, dstp, zc):
    f32 = jnp.float32
    k = pl.kernel(
        _edge_mp_body,
        out_type=[jax.ShapeDtypeStruct((NP, CH), f32)] * 2,
        mesh=_sc_mesh(),
        compiler_params=_sc_params(),
        scratch_types=_sc_mp_scratch(),
    )
    return k(y0, y1, s0, s1, srcp, dstp, zc)


def _edge_mp2_body(y_hbm, s_hbm, src_hbm, dst_hbm, zc_hbm, p0_hbm, p1_hbm,
                   isv0, isv1, idv0, idv1, rows0, rows1, srow0, srow1, acc,
                   semi0, semi1, semj0, semj1, semg0, semg1, sems0, sems1,
                   semw0, semw1):
    c = lax.axis_index("c")
    sid = lax.axis_index("s")
    wid = c * NS + sid
    pltpu.sync_copy(zc_hbm.at[pl.ds(sid * NROW, NROW)],
                    acc.at[pl.ds(sid * NROW, NROW)])
    plsc.subcore_barrier()
    nchunk = TPA // CHUNK
    _mp_pipeline(y_hbm, s_hbm, src_hbm, dst_hbm, acc,
                 (isv0, isv1), (idv0, idv1), (rows0, rows1), (srow0, srow1),
                 (semi0, semi1), (semj0, semj1), (semg0, semg1),
                 (sems0, sems1), (semw0, semw1),
                 nchunk, CHUNK, wid * TPA)
    plsc.subcore_barrier()

    @pl.when(c == 0)
    def _():
        pltpu.sync_copy(acc.at[pl.ds(sid * NROW, NROW)],
                        p0_hbm.at[pl.ds(sid * NROW, NROW)])

    @pl.when(c == 1)
    def _():
        pltpu.sync_copy(acc.at[pl.ds(sid * NROW, NROW)],
                        p1_hbm.at[pl.ds(sid * NROW, NROW)])


def _edge_mp2(y2, s2, srcp, dstp, zc):
    f32 = jnp.float32
    k = pl.kernel(
        _edge_mp2_body,
        out_type=[jax.ShapeDtypeStruct((NP, CH), f32)] * 2,
        mesh=_sc_mesh(),
        compiler_params=_sc_params(),
        scratch_types=_sc_mp_scratch(),
    )
    return k(y2, s2, srcp, dstp, zc)


def _edge_last_body(y4_hbm, bt_hbm, s4_hbm, src_hbm, dst_hbm, bins_hbm,
                    ytab, btab, sv, dv, s4v, bins, binsum):
    c = lax.axis_index("c")
    sid = lax.axis_index("s")
    wid = sid * NC + c
    base = wid * TPA

    def zb(i, _):
        bins[pl.ds(i * L, L)] = jnp.zeros((L,), jnp.float32)
        return 0

    lax.fori_loop(0, (L * N_GRAPHS) // L, zb, 0)
    pltpu.sync_copy(y4_hbm, ytab)
    pltpu.sync_copy(bt_hbm, btab)
    pltpu.sync_copy(src_hbm.at[pl.ds(base, TPA)], sv)
    pltpu.sync_copy(dst_hbm.at[pl.ds(base, TPA)], dv)
    pltpu.sync_copy(s4_hbm.at[pl.ds(base, TPA)], s4v)
    lanes = lax.iota(jnp.int32, L)

    def body(i, _):
        sl = pl.ds(i * L, L)
        isrc = sv[sl]
        idst = dv[sl]
        yv = plsc.load_gather(ytab, [isrc])
        bv = plsc.load_gather(btab, [idst])
        m = yv * s4v[sl]
        plsc.addupdate_scatter(bins, [lanes * N_GRAPHS + bv], m)
        return 0

    lax.fori_loop(0, TPA // L, body, 0)
    acc = bins[pl.ds(0, L)]
    for l in range(1, L):
        acc = acc + bins[pl.ds(l * L, L)]
    binsum[pl.ds(0, L)] = acc
    pltpu.sync_copy(binsum, bins_hbm.at[pl.ds(wid * N_GRAPHS, N_GRAPHS)])


def _edge_last(y4f, batch_i32, s4f, srcp, dstp):
    f32 = jnp.float32
    k = pl.kernel(
        _edge_last_body,
        out_type=[jax.ShapeDtypeStruct((NC * NS * N_GRAPHS,), f32)],
        mesh=_sc_mesh(),
        compiler_params=_sc_params(),
        scratch_types=[
            pltpu.VMEM((N_NODES,), f32),
            pltpu.VMEM((N_NODES,), jnp.int32),
            pltpu.VMEM((TPA,), jnp.int32),
            pltpu.VMEM((TPA,), jnp.int32),
            pltpu.VMEM((TPA,), f32),
            pltpu.VMEM((L * N_GRAPHS,), f32),
            pltpu.VMEM((N_GRAPHS,), f32),
        ],
    )
    return k(y4f, batch_i32, s4f, srcp, dstp)


def _pool_body(xf_ref, b_ref, bins_ref, wsc_ref, out_ref):
    i = pl.program_id(0)
    xf = xf_ref[...]
    b = b_ref[...]
    onehot = (b[:, 0:1] == lax.broadcasted_iota(jnp.int32, (1, N_GRAPHS), 1))
    ps = jnp.dot(onehot.astype(jnp.float32).T, xf,
                 preferred_element_type=jnp.float32, precision=lax.Precision.HIGHEST)
    zsum = jnp.dot(ps, wsc_ref[...],
                   preferred_element_type=jnp.float32, precision=lax.Precision.HIGHEST)

    @pl.when(i == 0)
    def _():
        bsum = jnp.sum(bins_ref[...], axis=0, keepdims=True)
        out_ref[...] = bsum * (INV * INV)

    out_ref[...] += zsum.T * INV


def _pool(xf, batch2d, bins, wsc4):
    return pl.pallas_call(
        _pool_body,
        grid=(N_NODES // BN,),
        in_specs=[
            pl.BlockSpec((BN, 296), lambda i: (i, 0)),
            pl.BlockSpec((BN, 128), lambda i: (i, 0)),
            pl.BlockSpec((NC * NS, N_GRAPHS), lambda i: (0, 0)),
            pl.BlockSpec((296, 1), lambda i: (0, 0)),
        ],
        out_specs=pl.BlockSpec((1, N_GRAPHS), lambda i: (0, 0)),
        out_shape=jax.ShapeDtypeStruct((1, N_GRAPHS), jnp.float32),
    )(xf, batch2d, bins, wsc4)


def _split3(w):
    return (w[:, :CH], w[:, CH:2 * CH],
            jnp.pad(w[:, 2 * CH:], ((0, 0), (0, CH - G2))))


def kernel(pos, x, edge_index, batch, params):
    f32 = jnp.float32
    src = edge_index[0].astype(jnp.int32)
    dst = edge_index[1].astype(jnp.int32)
    srcp = jnp.pad(src, (0, EP - N_EDGES))
    dstp = jnp.pad(dst, (0, EP - N_EDGES))
    px = pos[:, 0].astype(f32)
    py = pos[:, 1].astype(f32)
    pz = pos[:, 2].astype(f32)

    dx, dy, dz = _edge_vec(px, py, pz, srcp, dstp)
    emb, attr = _features(dx, dy, dz)
    zc = jnp.zeros((NP, CH), f32)

    aggs = None
    zprev = None
    for i in range(3):
        p = params[i]
        r0p = jnp.pad(p['R0'], ((0, 16 - N_BASIS), (0, 0)))
        w0, w1, w2 = _split3(p['W1'])
        r2s = _split3(p['R2'])
        wshs = _split3(p['Wsh'])
        if i == 0:
            y0, y1, y2, z = _node_mm(x, w0, w1, w2, p['Wsc'])
        else:
            y0, y1, y2, z = _node_mm(zprev, w0, w1, w2, p['Wsc'], aggs)
        s0, s1, s2 = _edge_chain(emb, attr, r0p, p['R1'], r2s, wshs)
        a0, a1 = _edge_mp(y0, y1, s0, s1, srcp, dstp, zc)
        p0, p1 = _edge_mp2(y2, s2, srcp, dstp, zc)
        aggs = (a0, a1, p0, p1)
        zprev = z

    p = params[3]
    r0p = jnp.pad(p['R0'], ((0, 16 - N_BASIS), (0, 0)))
    r2c = jnp.pad(p['R2'], ((0, 0), (0, CH - 1)))
    wshc = jnp.pad(p['Wsh'], ((0, 0), (0, CH - 1)))
    w4pad = jnp.pad(p['W1'], ((0, 0), (0, 127)))
    (s4a,) = _edge_chain(emb, attr, r0p, p['R1'], [r2c], [wshc])
    s4f = s4a[:, 0]
    y4mat, xf3 = _node_mm_last(zprev, aggs, w4pad)
    y4f = y4mat[:, 0]
    (bins,) = _edge_last(y4f, batch.astype(jnp.int32), s4f, srcp, dstp)
    bins2d = bins.reshape(NC * NS, N_GRAPHS)
    batch2d = jnp.pad(batch.astype(jnp.int32).reshape(N_NODES, 1),
                      ((0, 0), (0, 127)))
    out = _pool(xf3, batch2d, bins2d, p['Wsc'])
    return out.reshape(-1)
